# Initial kernel scaffold; baseline (speedup 1.0000x reference)
import jax
import jax.numpy as jnp
from jax.experimental import pallas as pl


def kernel(inputs, targets):
    raise NotImplementedError("write your pallas kernel here")


# speedup vs baseline: 92.8880x; 92.8880x over previous
import numpy as np
import jax
import jax.numpy as jnp
from jax.experimental import pallas as pl
from jax.experimental.pallas import tpu as pltpu

_BINS = 64
_N = 1000000
_OFFSET = 2.0 * np.pi / _BINS
_C0 = np.float32(-np.pi + _OFFSET / 2.0)
_C63 = np.float32(-np.pi + _OFFSET / 2.0 + 63 * _OFFSET)
_HALF_PI = np.float32(np.pi / 2.0)

_LANES = 128
_ROWS = 7816
_NP = _ROWS * _LANES
_BLOCK_R = 512
_GRID = (_ROWS + _BLOCK_R - 1) // _BLOCK_R


def _loss_kernel(in_ref, tg_ref, out_ref, sums_ref, cnts_ref):
    i = pl.program_id(0)

    @pl.when(i == 0)
    def _init():
        sums_ref[...] = jnp.zeros_like(sums_ref)
        cnts_ref[...] = jnp.zeros_like(cnts_ref)

    t0 = tg_ref[0]
    t1 = tg_ref[1]
    t2 = tg_ref[2]

    acc_u = jnp.zeros_like(t0)
    acc_v = jnp.zeros_like(t0)
    for c in range(3):
        a = in_ref[c]
        t = tg_ref[c]
        sd = jnp.sin((a - t) * 0.5)
        cs = jnp.cos((a + t) * 0.5)
        u = sd * sd
        acc_u = acc_u + u
        acc_v = acc_v + cs * cs * u
    r = 2.0 * (jnp.sqrt(acc_v) + jnp.sqrt(acc_u - acc_v))

    phi_hi = jnp.abs(t0 - _C63) > jnp.abs(t0 - _C0)
    psi_hi = jnp.abs(t1 - _C63) > jnp.abs(t1 - _C0)
    om_hi = jnp.abs(t2) > _HALF_PI

    row = jax.lax.broadcasted_iota(jnp.int32, (_BLOCK_R, _LANES), 0)
    lane = jax.lax.broadcasted_iota(jnp.int32, (_BLOCK_R, _LANES), 1)
    sid = (i * _BLOCK_R + row) * _LANES + lane
    valid = sid < _N

    for b in range(8):
        m = valid
        m = m & (phi_hi if (b & 4) else ~phi_hi)
        m = m & (psi_hi if (b & 2) else ~psi_hi)
        m = m & (om_hi if (b & 1) else ~om_hi)
        sums_ref[b, :] = sums_ref[b, :] + jnp.sum(
            jnp.where(m, r, 0.0), axis=0)
        cnts_ref[b, :] = cnts_ref[b, :] + jnp.sum(
            m.astype(jnp.float32), axis=0)

    @pl.when(i == _GRID - 1)
    def _finish():
        s = jnp.sum(sums_ref[...], axis=1)
        c = jnp.sum(cnts_ref[...], axis=1)
        total = jnp.sum(jnp.where(c > 0.0, s / c, 0.0))
        out_ref[...] = jnp.reshape(total / np.float32(_N), (1, 1))


def _prep(x):
    xp = jnp.pad(x, ((0, _NP - _N), (0, 0)))
    return xp.T.reshape(3, _ROWS, _LANES)


def kernel(inputs, targets):
    ai = _prep(inputs)
    at = _prep(targets)
    spec = pl.BlockSpec((3, _BLOCK_R, _LANES), lambda i: (0, i, 0))
    out = pl.pallas_call(
        _loss_kernel,
        grid=(_GRID,),
        in_specs=[spec, spec],
        out_specs=pl.BlockSpec((1, 1), lambda i: (0, 0)),
        out_shape=jax.ShapeDtypeStruct((1, 1), jnp.float32),
        scratch_shapes=[
            pltpu.VMEM((8, _LANES), jnp.float32),
            pltpu.VMEM((8, _LANES), jnp.float32),
        ],
    )(ai, at)
    return out[0, 0]


# speedup vs baseline: 153.3398x; 1.6508x over previous
import numpy as np
import jax
import jax.numpy as jnp
from jax.experimental import pallas as pl
from jax.experimental.pallas import tpu as pltpu

_BINS = 64
_N = 1000000
_OFFSET = 2.0 * np.pi / _BINS
_C0 = np.float32(-np.pi + _OFFSET / 2.0)
_C63 = np.float32(-np.pi + _OFFSET / 2.0 + 63 * _OFFSET)
_HALF_PI = np.float32(np.pi / 2.0)

_LANES = 128
_ROWS = 7816
_NP = _ROWS * _LANES
_BLOCK_R = 512
_GRID = (_ROWS + _BLOCK_R - 1) // _BLOCK_R

_INV2PI = np.float32(1.0 / (2.0 * np.pi))
_MAGIC = np.float32(12582912.0)
_P1 = np.float32(6.28125)
_P2 = np.float32(2.0 * np.pi - 6.28125)
_SIN_C = tuple(np.float32(v) for v in (
    0.9999997, -0.16666578, 0.008332558, -0.00019812577,
    2.7040517e-06, -2.0534266e-08))
_COS_C = tuple(np.float32(v) for v in (
    1.0, -0.4999999, 0.041666523, -0.0013887971,
    2.4773424e-05, -2.7113373e-07, 1.7369133e-09))


def _reduce_2pi(x):
    n = jax.lax.round(x * _INV2PI, jax.lax.RoundingMethod.TO_NEAREST_EVEN)
    return (x - n * _P1) - n * _P2


def _fast_sin(x):
    r = _reduce_2pi(x)
    r2 = r * r
    p = _SIN_C[5]
    for c in (_SIN_C[4], _SIN_C[3], _SIN_C[2], _SIN_C[1], _SIN_C[0]):
        p = p * r2 + c
    return r * p


def _fast_cos(x):
    r = _reduce_2pi(x)
    r2 = r * r
    p = _COS_C[6]
    for c in (_COS_C[5], _COS_C[4], _COS_C[3], _COS_C[2], _COS_C[1], _COS_C[0]):
        p = p * r2 + c
    return p


def _loss_kernel(in_ref, tg_ref, out_ref, sums_ref, cnts_ref):
    i = pl.program_id(0)

    @pl.when(i == 0)
    def _init():
        sums_ref[...] = jnp.zeros_like(sums_ref)
        cnts_ref[...] = jnp.zeros_like(cnts_ref)

    t0 = tg_ref[0]
    t1 = tg_ref[1]
    t2 = tg_ref[2]

    acc_u = jnp.zeros_like(t0)
    acc_v = jnp.zeros_like(t0)
    for c in range(3):
        a = in_ref[c]
        t = tg_ref[c]
        sd = _fast_sin((a - t) * 0.5)
        cs = _fast_cos((a + t) * 0.5)
        u = sd * sd
        acc_u = acc_u + u
        acc_v = acc_v + cs * cs * u
    r = 2.0 * (jnp.sqrt(acc_v) + jnp.sqrt(acc_u - acc_v))

    phi_hi = jnp.abs(t0 - _C63) > jnp.abs(t0 - _C0)
    psi_hi = jnp.abs(t1 - _C63) > jnp.abs(t1 - _C0)
    om_hi = jnp.abs(t2) > _HALF_PI

    row = jax.lax.broadcasted_iota(jnp.int32, (_BLOCK_R, _LANES), 0)
    lane = jax.lax.broadcasted_iota(jnp.int32, (_BLOCK_R, _LANES), 1)
    sid = (i * _BLOCK_R + row) * _LANES + lane
    valid = sid < _N

    for b in range(8):
        m = valid
        m = m & (phi_hi if (b & 4) else ~phi_hi)
        m = m & (psi_hi if (b & 2) else ~psi_hi)
        m = m & (om_hi if (b & 1) else ~om_hi)
        sums_ref[b, :] = sums_ref[b, :] + jnp.sum(
            jnp.where(m, r, 0.0), axis=0)
        cnts_ref[b, :] = cnts_ref[b, :] + jnp.sum(
            m.astype(jnp.float32), axis=0)

    @pl.when(i == _GRID - 1)
    def _finish():
        s = jnp.sum(sums_ref[...], axis=1)
        c = jnp.sum(cnts_ref[...], axis=1)
        total = jnp.sum(jnp.where(c > 0.0, s / c, 0.0))
        out_ref[...] = jnp.reshape(total / np.float32(_N), (1, 1))


def _prep(x):
    xp = jnp.pad(x, ((0, _NP - _N), (0, 0)))
    return xp.T.reshape(3, _ROWS, _LANES)


def kernel(inputs, targets):
    ai = _prep(inputs)
    at = _prep(targets)
    spec = pl.BlockSpec((3, _BLOCK_R, _LANES), lambda i: (0, i, 0))
    out = pl.pallas_call(
        _loss_kernel,
        grid=(_GRID,),
        in_specs=[spec, spec],
        out_specs=pl.BlockSpec((1, 1), lambda i: (0, 0)),
        out_shape=jax.ShapeDtypeStruct((1, 1), jnp.float32),
        scratch_shapes=[
            pltpu.VMEM((8, _LANES), jnp.float32),
            pltpu.VMEM((8, _LANES), jnp.float32),
        ],
    )(ai, at)
    return out[0, 0]
